# Initial kernel scaffold; baseline (speedup 1.0000x reference)
import jax
import jax.numpy as jnp
from jax.experimental import pallas as pl


def kernel(normalized_matrix, W1, b1, W2, b2):
    raise NotImplementedError("write your pallas kernel here")


# speedup vs baseline: 1.3830x; 1.3830x over previous
import jax
import jax.numpy as jnp
from jax.experimental import pallas as pl
from jax.experimental.pallas import tpu as pltpu

L = 16384
S = 2048
D = 16
H = 16
BR = 1024


def _body(w1_ref, b1_ref, w2_ref, b2_ref, x_ref, out_ref):
    cols = jax.lax.broadcasted_iota(jnp.int32, (S, H), 0).astype(jnp.float32)
    h = jax.nn.relu(cols * w1_ref[:, :] + b1_ref[:, :])
    P = jnp.dot(h, w2_ref[:, :], preferred_element_type=jnp.float32) + b2_ref[:, :]

    x = x_ref[:, :]
    mask = (x != 0.0).astype(jnp.float32)
    cnt = jnp.sum(mask, axis=1, keepdims=True)
    summed = jnp.dot(mask, P, preferred_element_type=jnp.float32)
    out = summed / jnp.maximum(cnt, 1.0)
    out_ref[:, :] = jnp.where(cnt > 0.0, out, jnp.zeros_like(out))


def kernel(normalized_matrix, W1, b1, W2, b2):
    w1 = W1.reshape(1, H)
    b1r = b1.reshape(1, H)
    b2r = b2.reshape(1, D)
    grid = (L // BR,)
    return pl.pallas_call(
        _body,
        grid=grid,
        in_specs=[
            pl.BlockSpec((1, H), lambda i: (0, 0)),
            pl.BlockSpec((1, H), lambda i: (0, 0)),
            pl.BlockSpec((H, D), lambda i: (0, 0)),
            pl.BlockSpec((1, D), lambda i: (0, 0)),
            pl.BlockSpec((BR, S), lambda i: (i, 0)),
        ],
        out_specs=pl.BlockSpec((BR, D), lambda i: (i, 0)),
        out_shape=jax.ShapeDtypeStruct((L, D), jnp.float32),
    )(w1, b1r, W2, b2r, normalized_matrix)
